# baseline (speedup 1.0000x reference)
import jax
import jax.numpy as jnp
from jax.experimental import pallas as pl
from jax.experimental.pallas import tpu as pltpu

_NUM_EXPERTS = 64
_TOPK = 8
_BLK = 1024


def _router_body(x_ref, w_ref, gates_ref, idx_ref, wt_ref):
    @pl.when(pl.program_id(0) == 0)
    def _():
        wt_ref[...] = w_ref[...].T

    logits = jax.lax.dot_general(
        x_ref[...], wt_ref[...],
        dimension_numbers=(((1,), (0,)), ((), ())),
        preferred_element_type=jnp.float32,
    )
    lane = jax.lax.broadcasted_iota(jnp.uint32, logits.shape, 1)
    ex = jnp.exp(logits - jnp.max(logits, axis=1, keepdims=True))
    s = ex / jnp.sum(ex, axis=1, keepdims=True)
    sbits = jax.lax.bitcast_convert_type(s, jnp.uint32)
    keyu = (sbits & jnp.uint32(0xFFFFFFC0)) + jnp.uint32(0x00800000) \
        + (jnp.uint32(_NUM_EXPERTS - 1) - lane)
    key = jax.lax.bitcast_convert_type(keyu, jnp.float32)
    picks = []
    for _ in range(_TOPK):
        m = jnp.max(key, axis=1, keepdims=True)
        picks.append(m)
        key = jnp.where(key == m, 0.0, key)
    p = jax.lax.bitcast_convert_type(
        jnp.concatenate(picks, axis=1), jnp.uint32)
    idx = jnp.uint32(_NUM_EXPERTS - 1) - (p & jnp.uint32(_NUM_EXPERTS - 1))
    v = jax.lax.bitcast_convert_type(
        (p - jnp.uint32(0x00800000)) & jnp.uint32(0xFFFFFFC0), jnp.float32)
    gates = v / jnp.sum(v, axis=1, keepdims=True)
    gates_ref[...] = gates.T
    idx_ref[...] = idx.astype(jnp.int32).T


@jax.jit
def kernel(X, W):
    B, T, C = X.shape
    tok = B * T
    Xf = X.reshape(tok, C)
    grid = (tok // _BLK,)
    gates, idx = pl.pallas_call(
        _router_body,
        grid=grid,
        in_specs=[
            pl.BlockSpec((_BLK, C), lambda i: (i, 0)),
            pl.BlockSpec((_NUM_EXPERTS, C), lambda i: (0, 0)),
        ],
        out_specs=[
            pl.BlockSpec((_TOPK, _BLK), lambda i: (0, i)),
            pl.BlockSpec((_TOPK, _BLK), lambda i: (0, i)),
        ],
        out_shape=[
            jax.ShapeDtypeStruct((_TOPK, tok), jnp.float32),
            jax.ShapeDtypeStruct((_TOPK, tok), jnp.int32),
        ],
        scratch_shapes=[
            pltpu.VMEM((C, _NUM_EXPERTS), jnp.float32),
        ],
        compiler_params=pltpu.CompilerParams(
            dimension_semantics=("arbitrary",),
        ),
    )(Xf, W)
    return (gates.T.reshape(B, T, _TOPK), idx.T.reshape(B, T, _TOPK))


# speedup vs baseline: 1.0348x; 1.0348x over previous
import jax
import jax.numpy as jnp
from jax.experimental import pallas as pl
from jax.experimental.pallas import tpu as pltpu

_NUM_EXPERTS = 64
_TOPK = 8
_BLK = 1024


def _router_body(x_ref, w_ref, gates_ref, idx_ref, wt_ref):
    @pl.when(pl.program_id(0) == 0)
    def _():
        wt_ref[...] = w_ref[...].T

    logits = jax.lax.dot_general(
        x_ref[...], wt_ref[...],
        dimension_numbers=(((1,), (0,)), ((), ())),
        preferred_element_type=jnp.float32,
    )
    lane = jax.lax.broadcasted_iota(jnp.uint32, logits.shape, 1)
    ex = jnp.exp(logits - jnp.max(logits, axis=1, keepdims=True))
    s = ex / jnp.sum(ex, axis=1, keepdims=True)
    sbits = jax.lax.bitcast_convert_type(s, jnp.uint32)
    keyu = (sbits & jnp.uint32(0xFFFFFFC0)) + jnp.uint32(0x00800000) \
        + (jnp.uint32(_NUM_EXPERTS - 1) - lane)
    key = jax.lax.bitcast_convert_type(keyu, jnp.float32)
    picks = []
    for _ in range(_TOPK):
        m = jnp.max(key, axis=1, keepdims=True)
        picks.append(m)
        key = jnp.where(key == m, 0.0, key)
    p = jax.lax.bitcast_convert_type(
        jnp.concatenate(picks, axis=1).T, jnp.uint32)
    idx = jnp.uint32(_NUM_EXPERTS - 1) - (p & jnp.uint32(_NUM_EXPERTS - 1))
    v = jax.lax.bitcast_convert_type(
        (p - jnp.uint32(0x00800000)) & jnp.uint32(0xFFFFFFC0), jnp.float32)
    gates_ref[...] = v / jnp.sum(v, axis=0, keepdims=True)
    idx_ref[...] = idx.astype(jnp.int32)


@jax.jit
def kernel(X, W):
    B, T, C = X.shape
    tok = B * T
    Xf = X.reshape(tok, C)
    grid = (tok // _BLK,)
    gates, idx = pl.pallas_call(
        _router_body,
        grid=grid,
        in_specs=[
            pl.BlockSpec((_BLK, C), lambda i: (i, 0)),
            pl.BlockSpec((_NUM_EXPERTS, C), lambda i: (0, 0)),
        ],
        out_specs=[
            pl.BlockSpec((_TOPK, _BLK), lambda i: (0, i)),
            pl.BlockSpec((_TOPK, _BLK), lambda i: (0, i)),
        ],
        out_shape=[
            jax.ShapeDtypeStruct((_TOPK, tok), jnp.float32),
            jax.ShapeDtypeStruct((_TOPK, tok), jnp.int32),
        ],
        scratch_shapes=[
            pltpu.VMEM((C, _NUM_EXPERTS), jnp.float32),
        ],
        compiler_params=pltpu.CompilerParams(
            dimension_semantics=("arbitrary",),
        ),
    )(Xf, W)
    return (gates.T.reshape(B, T, _TOPK), idx.T.reshape(B, T, _TOPK))
